# baseline (speedup 1.0000x reference)
import functools

import jax
import jax.numpy as jnp
from jax import lax
from jax.experimental import pallas as pl
from jax.experimental.pallas import tpu as pltpu
from jax.experimental.pallas import tpu_sc as plsc

BATCH = 16384
EMBED = 64
HIDDEN = 128

_NC, _NS = 2, 16
_NW = _NC * _NS
_BPW = BATCH // _NW
_CHUNK = 128
_NCHUNK = _BPW // _CHUNK

@functools.lru_cache(maxsize=1)
def _make_gather_sc():
    mesh = plsc.VectorSubcoreMesh(core_axis_name="c", subcore_axis_name="s")

    @functools.partial(
        pl.kernel,
        mesh=mesh,
        compiler_params=pltpu.CompilerParams(use_tc_tiling_on_sc=False),
        out_type=[
            jax.ShapeDtypeStruct((BATCH, EMBED), jnp.float32),
            jax.ShapeDtypeStruct((BATCH, EMBED), jnp.float32),
        ],
        scratch_types=[
            pltpu.VMEM((_NCHUNK, _CHUNK), jnp.int32),
            pltpu.VMEM((_NCHUNK, _CHUNK), jnp.int32),
            pltpu.VMEM((_BPW, EMBED), jnp.float32),
            pltpu.VMEM((_BPW, EMBED), jnp.float32),
            pltpu.SemaphoreType.DMA,
            pltpu.SemaphoreType.DMA,
        ],
    )
    def _gather_sc(uidx_hbm, bidx_hbm, utab_hbm, btab_hbm, u_out, v_out,
                   uidx_v, bidx_v, urows, vrows, usem, vsem):
        wid = lax.axis_index("s") * _NC + lax.axis_index("c")
        crow = wid * _NCHUNK
        pltpu.sync_copy(uidx_hbm.at[pl.ds(crow, _NCHUNK)], uidx_v)
        pltpu.sync_copy(bidx_hbm.at[pl.ds(crow, _NCHUNK)], bidx_v)
        copies = []
        for j in range(_NCHUNK):
            dst = pl.ds(j * _CHUNK, _CHUNK)
            copies.append(pltpu.async_copy(
                utab_hbm.at[uidx_v.at[j]], urows.at[dst], usem))
            copies.append(pltpu.async_copy(
                btab_hbm.at[bidx_v.at[j]], vrows.at[dst], vsem))
        for c in copies:
            c.wait()
        base = wid * _BPW
        pltpu.sync_copy(urows, u_out.at[pl.ds(base, _BPW)])
        pltpu.sync_copy(vrows, v_out.at[pl.ds(base, _BPW)])

    return _gather_sc


def _mlp_body(u_ref, v_ref, w1a_ref, w1b_ref, b1_ref, w2_ref, b2_ref, o_ref):
    h = (jnp.dot(u_ref[...], w1a_ref[...], preferred_element_type=jnp.float32,
                 precision=lax.Precision.HIGHEST)
         + jnp.dot(v_ref[...], w1b_ref[...], preferred_element_type=jnp.float32,
                   precision=lax.Precision.HIGHEST)
         + b1_ref[...])
    h = jnp.where(h >= 0, h, 0.01 * h)
    raw = jnp.sum(h * w2_ref[...], axis=1, keepdims=True) + b2_ref[...]
    o_ref[...] = 1.0 + 4.0 * jax.nn.sigmoid(raw)


_BM = 2048


def _mlp(u, v, w1a, w1b, b1, w2, b2):
    grid = (BATCH // _BM,)
    return pl.pallas_call(
        _mlp_body,
        grid=grid,
        in_specs=[
            pl.BlockSpec((_BM, EMBED), lambda i: (i, 0)),
            pl.BlockSpec((_BM, EMBED), lambda i: (i, 0)),
            pl.BlockSpec((EMBED, HIDDEN), lambda i: (0, 0)),
            pl.BlockSpec((EMBED, HIDDEN), lambda i: (0, 0)),
            pl.BlockSpec((1, HIDDEN), lambda i: (0, 0)),
            pl.BlockSpec((1, HIDDEN), lambda i: (0, 0)),
            pl.BlockSpec((1, 1), lambda i: (0, 0)),
        ],
        out_specs=pl.BlockSpec((_BM, 1), lambda i: (i, 0)),
        out_shape=jax.ShapeDtypeStruct((BATCH, 1), jnp.float32),
    )(u, v, w1a, w1b, b1, w2, b2)


def kernel(user_id, book_id, user_table, book_table, W1, b1, W2, b2):
    uidx = user_id.astype(jnp.int32).reshape(_NW * _NCHUNK, _CHUNK)
    bidx = book_id.astype(jnp.int32).reshape(_NW * _NCHUNK, _CHUNK)
    u, v = _make_gather_sc()(uidx, bidx, user_table, book_table)
    w1t = W1.T
    return _mlp(u, v, w1t[:EMBED], w1t[EMBED:], b1.reshape(1, HIDDEN),
                W2.reshape(1, HIDDEN), b2.reshape(1, 1))


# speedup vs baseline: 5.1675x; 5.1675x over previous
import functools

import jax
import jax.numpy as jnp
from jax import lax
from jax.experimental import pallas as pl
from jax.experimental.pallas import tpu as pltpu
from jax.experimental.pallas import tpu_sc as plsc

BATCH = 16384
EMBED = 64
HIDDEN = 128
NROWS = 1000000

_NC, _NS = 2, 16
_NW = _NC * _NS
_BPW = BATCH // _NW
NBLK = (NROWS + 127) // 128
RS = 8
RR = 16

_i32 = jnp.int32
_GDN = lax.GatherDimensionNumbers(offset_dims=(), collapsed_slice_dims=(0,),
                                  start_index_map=(0,))


def _rot_gather(v, idx):
    return lax.gather(v, idx[:, None], _GDN, slice_sizes=(1,),
                      mode=lax.GatherScatterMode.PROMISE_IN_BOUNDS)


@functools.lru_cache(maxsize=1)
def _make_gather_sc():
    mesh = plsc.VectorSubcoreMesh(core_axis_name="c", subcore_axis_name="s")

    @functools.partial(
        pl.kernel,
        mesh=mesh,
        out_type=[
            jax.ShapeDtypeStruct((BATCH, EMBED), jnp.float32),
            jax.ShapeDtypeStruct((BATCH, EMBED), jnp.float32),
        ],
        scratch_types=[
            pltpu.VMEM((_BPW,), _i32),
            pltpu.VMEM((_BPW,), _i32),
            pltpu.VMEM((RS, EMBED, 128), jnp.float32),
            pltpu.VMEM((RR, EMBED), jnp.float32),
            pltpu.SMEM((_BPW + 1,), _i32),
            pltpu.SMEM((_BPW + 1,), _i32),
            pltpu.SemaphoreType.DMA((RS,)),
            pltpu.SemaphoreType.DMA((RR,)),
        ],
    )
    def _gather_sc(seu_hbm, sou_hbm, seb_hbm, sob_hbm,
                   utabt_hbm, btabt_hbm, uedge_hbm, bedge_hbm,
                   u_out, v_out,
                   sidx_v, spos_v, slabs, rowring, blist_s, rstart_s,
                   sem_slab, sem_row):
        wid = lax.axis_index("s") * _NC + lax.axis_index("c")
        base = wid * _BPW
        iota = lax.iota(_i32, 16)

        def one_table(se_hbm, so_hbm, tabt_hbm, edge_hbm, out_hbm):
            pltpu.sync_copy(se_hbm.at[pl.ds(base, _BPW)], sidx_v)
            pltpu.sync_copy(so_hbm.at[pl.ds(base, _BPW)], spos_v)

            def p1_body(g, carry):
                nb, prev = carry
                ev = sidx_v[pl.ds(g * 16, 16)]
                bv = jnp.right_shift(ev, 7)
                for j in range(16):
                    b = bv[j]
                    blist_s[nb] = b
                    rstart_s[nb] = g * 16 + j
                    nb = nb + (b != prev).astype(_i32)
                    prev = b
                return (nb, prev)

            nb, _ = lax.fori_loop(0, _BPW // 16, p1_body,
                                  (_i32(0), _i32(-1)))
            rstart_s[nb] = _BPW

            def issue_slab(p, slot):
                g = blist_s[p]
                safe = jnp.minimum(g, NBLK - 2) * 128

                @pl.when(g < NBLK - 1)
                def _():
                    pltpu.async_copy(tabt_hbm.at[:, pl.ds(safe, 128)],
                                     slabs.at[slot], sem_slab.at[slot])

                @pl.when(g == NBLK - 1)
                def _():
                    pltpu.async_copy(edge_hbm, slabs.at[slot],
                                     sem_slab.at[slot])

            def pro_body(p, c):
                @pl.when(p < nb)
                def _():
                    issue_slab(p, p)
                return c

            lax.fori_loop(0, RS, pro_body, _i32(0))

            def wave(w, jg):
                slot = lax.rem(w, RS)
                pltpu.make_async_copy(
                    tabt_hbm.at[:, pl.ds(0, 128)], slabs.at[0],
                    sem_slab.at[slot]).wait()
                g = blist_s[w]
                start = g * 128
                rs = rstart_s[w]
                re = rstart_s[w + 1]

                def group(gg, jg):
                    ev = sidx_v[pl.ds(gg * 16, 16)]
                    pv = spos_v[pl.ds(gg * 16, 16)]
                    for j in range(16):
                        h = gg * 16 + j
                        pred = jnp.logical_and(h >= rs, h < re)
                        jr = lax.rem(jg, RR)

                        @pl.when(jnp.logical_and(pred, jg >= RR))
                        def _():
                            pltpu.make_async_copy(
                                rowring.at[0], out_hbm.at[0],
                                sem_row.at[jr]).wait()

                        @pl.when(pred)
                        def _():
                            e = ev[j]
                            pos = pv[j]
                            l = e - start
                            l15 = l & 15
                            lbase = (l >> 4) << 4
                            ivec = iota + l15

                            def chunk(kk, c):
                                acc = jnp.zeros((16,), jnp.float32)
                                row0 = kk * 16
                                for j2 in range(16):
                                    v = slabs[slot, row0 + j2,
                                              pl.ds(lbase, 16)]
                                    rot = _rot_gather(v, (ivec - j2) & 15)
                                    acc = jnp.where(iota == j2, rot, acc)
                                rowring[jr, pl.ds(row0, 16)] = acc
                                return c

                            lax.fori_loop(0, EMBED // 16, chunk, _i32(0))
                            pltpu.async_copy(rowring.at[jr],
                                             out_hbm.at[pos],
                                             sem_row.at[jr])

                        jg = jg + pred.astype(_i32)
                    return jg

                g0 = rs >> 4
                g1 = (re + 15) >> 4
                jg = lax.fori_loop(g0, g1, group, jg)

                @pl.when(w + RS < nb)
                def _():
                    issue_slab(w + RS, slot)

                return jg

            jg = lax.fori_loop(0, nb, wave, _i32(0))

            def drain_body(d, c):
                @pl.when(d < jnp.minimum(jg, RR))
                def _():
                    pltpu.make_async_copy(
                        rowring.at[0], out_hbm.at[0], sem_row.at[d]).wait()
                return c

            lax.fori_loop(0, RR, drain_body, _i32(0))

        one_table(seu_hbm, sou_hbm, utabt_hbm, uedge_hbm, u_out)
        one_table(seb_hbm, sob_hbm, btabt_hbm, bedge_hbm, v_out)

    return _gather_sc


def _mlp_body(u_ref, v_ref, w1a_ref, w1b_ref, b1_ref, w2_ref, b2_ref, o_ref):
    h = (jnp.dot(u_ref[...], w1a_ref[...], preferred_element_type=jnp.float32,
                 precision=lax.Precision.HIGHEST)
         + jnp.dot(v_ref[...], w1b_ref[...], preferred_element_type=jnp.float32,
                   precision=lax.Precision.HIGHEST)
         + b1_ref[...])
    h = jnp.where(h >= 0, h, 0.01 * h)
    raw = jnp.sum(h * w2_ref[...], axis=1, keepdims=True) + b2_ref[...]
    o_ref[...] = 1.0 + 4.0 * jax.nn.sigmoid(raw)


_BM = 2048


def _mlp(u, v, w1a, w1b, b1, w2, b2):
    grid = (BATCH // _BM,)
    return pl.pallas_call(
        _mlp_body,
        grid=grid,
        in_specs=[
            pl.BlockSpec((_BM, EMBED), lambda i: (i, 0)),
            pl.BlockSpec((_BM, EMBED), lambda i: (i, 0)),
            pl.BlockSpec((EMBED, HIDDEN), lambda i: (0, 0)),
            pl.BlockSpec((EMBED, HIDDEN), lambda i: (0, 0)),
            pl.BlockSpec((1, HIDDEN), lambda i: (0, 0)),
            pl.BlockSpec((1, HIDDEN), lambda i: (0, 0)),
            pl.BlockSpec((1, 1), lambda i: (0, 0)),
        ],
        out_specs=pl.BlockSpec((_BM, 1), lambda i: (i, 0)),
        out_shape=jax.ShapeDtypeStruct((BATCH, 1), jnp.float32),
    )(u, v, w1a, w1b, b1, w2, b2)


def kernel(user_id, book_id, user_table, book_table, W1, b1, W2, b2):
    uidx = user_id.astype(_i32)
    bidx = book_id.astype(_i32)
    pos = jnp.arange(BATCH, dtype=_i32)
    seu, sou = lax.sort_key_val(uidx, pos)
    seb, sob = lax.sort_key_val(bidx, pos)
    ut = user_table.T
    bt = book_table.T
    padw = NBLK * 128 - NROWS
    uedge = jnp.pad(ut[:, (NBLK - 1) * 128:], ((0, 0), (0, padw)))
    bedge = jnp.pad(bt[:, (NBLK - 1) * 128:], ((0, 0), (0, padw)))
    u, v = _make_gather_sc()(seu, sou, seb, sob, ut, bt, uedge, bedge)
    w1t = W1.T
    return _mlp(u, v, w1t[:EMBED], w1t[EMBED:], b1.reshape(1, HIDDEN),
                W2.reshape(1, HIDDEN), b2.reshape(1, 1))
